# Initial kernel scaffold; baseline (speedup 1.0000x reference)
import functools

import jax
import jax.numpy as jnp
from jax import lax
from jax.experimental import pallas as pl
from jax.experimental.pallas import tpu as pltpu
from jax.experimental.pallas import tpu_sc as plsc

MAX_ID = 99999
DIM = 64
WALK_LEN = 5
WALK_NUM = 2
WIN = 2
NUM_NEGS = 5
BATCH = 1024
L = WALK_LEN + 1

_PAIR_SRC = []
_PAIR_POS = []
for _i in range(L):
    for _j in range(max(0, _i - WIN), min(L, _i + WIN + 1)):
        if _j != _i:
            _PAIR_SRC.append(_i)
            _PAIR_POS.append(_j)
NUM_PAIRS = len(_PAIR_SRC)

N = WALK_NUM * BATCH * NUM_PAIRS
CHUNK = 128
NCHUNKS = N // CHUNK
NW = 32
CPW = NCHUNKS // NW
OPW = BATCH // NW
NLANES = 16
NGROUPS = CHUNK // NLANES

_mesh = plsc.VectorSubcoreMesh(core_axis_name="c", subcore_axis_name="s")


@functools.partial(
    pl.kernel,
    out_type=[
        jax.ShapeDtypeStruct((BATCH, DIM), jnp.float32),
        jax.ShapeDtypeStruct((BATCH, DIM), jnp.float32),
        jax.ShapeDtypeStruct((NCHUNKS, 6, CHUNK), jnp.float32),
    ],
    mesh=_mesh,
    scratch_types=[
        pltpu.VMEM((7, CHUNK), jnp.int32),
        pltpu.VMEM((CHUNK, DIM), jnp.float32),
        pltpu.VMEM((CHUNK, DIM), jnp.float32),
        pltpu.VMEM((NUM_NEGS, CHUNK, DIM), jnp.float32),
        pltpu.VMEM((6, CHUNK), jnp.float32),
        pltpu.VMEM((OPW,), jnp.int32),
        pltpu.VMEM((OPW, DIM), jnp.float32),
        pltpu.VMEM((OPW, DIM), jnp.float32),
        pltpu.SemaphoreType.DMA,
    ],
)
def _sc_deepwalk(tgt_hbm, ctx_hbm, idx_hbm, inp_hbm,
                 out_emb_hbm, out_ctx_hbm, logits_hbm,
                 idx_v, emb_v, pos_v, neg_v, lg_v, oidx_v, oemb_v, octx_v,
                 sem):
    nc = _mesh.num_cores
    wid = lax.axis_index("s") * nc + lax.axis_index("c")

    ob = wid * OPW
    pltpu.sync_copy(inp_hbm.at[pl.ds(ob, OPW)], oidx_v)
    h1 = pltpu.async_copy(tgt_hbm.at[oidx_v], oemb_v, sem)
    h2 = pltpu.async_copy(ctx_hbm.at[oidx_v], octx_v, sem)
    h1.wait()
    h2.wait()
    pltpu.sync_copy(oemb_v, out_emb_hbm.at[pl.ds(ob, OPW)])
    pltpu.sync_copy(octx_v, out_ctx_hbm.at[pl.ds(ob, OPW)])

    def chunk_body(ci, _):
        c = wid * CPW + ci
        pltpu.sync_copy(idx_hbm.at[c], idx_v)
        hs = [
            pltpu.async_copy(tgt_hbm.at[idx_v.at[0]], emb_v, sem),
            pltpu.async_copy(ctx_hbm.at[idx_v.at[1]], pos_v, sem),
        ]
        for k in range(NUM_NEGS):
            hs.append(
                pltpu.async_copy(ctx_hbm.at[idx_v.at[2 + k]], neg_v.at[k], sem))
        for h in hs:
            h.wait()

        for g in range(NGROUPS):
            rows = lax.iota(jnp.int32, NLANES) + g * NLANES

            def dstep(d, accs):
                col = jnp.full((NLANES,), 0, jnp.int32) + d
                ve = plsc.load_gather(emb_v, [rows, col])
                vp = plsc.load_gather(pos_v, [rows, col])
                out = [accs[0] + ve * vp]
                for k in range(NUM_NEGS):
                    vn = plsc.load_gather(neg_v.at[k], [rows, col])
                    out.append(accs[k + 1] + ve * vn)
                return tuple(out)

            zero = jnp.zeros((NLANES,), jnp.float32)
            accs = lax.fori_loop(0, DIM, dstep, (zero,) * 6)
            for j in range(6):
                lg_v[j, pl.ds(g * NLANES, NLANES)] = accs[j]

        pltpu.sync_copy(lg_v, logits_hbm.at[c])
        return 0

    lax.fori_loop(0, CPW, chunk_body, 0)


def _tc_loss_body(lg_ref, loss_ref, mrr_ref):
    x = lg_ref[...]
    pos = x[:, 0:1, :]
    negs = x[:, 1:, :]
    sp_pos = jnp.maximum(-pos, 0.0) + jnp.log1p(jnp.exp(-jnp.abs(pos)))
    sp_neg = jnp.maximum(negs, 0.0) + jnp.log1p(jnp.exp(-jnp.abs(negs)))
    loss_ref[0, 0] = jnp.sum(sp_pos) + jnp.sum(sp_neg)
    rank = 1.0 + jnp.sum((negs >= pos).astype(jnp.float32), axis=1)
    mrr_ref[0, 0] = jnp.sum(1.0 / rank) * (1.0 / N)


_tc_loss = pl.pallas_call(
    _tc_loss_body,
    out_shape=[
        jax.ShapeDtypeStruct((1, 1), jnp.float32),
        jax.ShapeDtypeStruct((1, 1), jnp.float32),
    ],
    out_specs=[
        pl.BlockSpec(memory_space=pltpu.SMEM),
        pl.BlockSpec(memory_space=pltpu.SMEM),
    ],
)


def kernel(inputs, paths, negs, target_table, context_table):
    src_i = jnp.asarray(_PAIR_SRC, jnp.int32)
    pos_i = jnp.asarray(_PAIR_POS, jnp.int32)
    src = jnp.take(paths, src_i, axis=2).reshape(N).astype(jnp.int32)
    pos = jnp.take(paths, pos_i, axis=2).reshape(N).astype(jnp.int32)
    ng = negs.reshape(N, NUM_NEGS).astype(jnp.int32)
    idx_all = jnp.concatenate(
        [
            src.reshape(NCHUNKS, 1, CHUNK),
            pos.reshape(NCHUNKS, 1, CHUNK),
            ng.reshape(NCHUNKS, CHUNK, NUM_NEGS).transpose(0, 2, 1),
        ],
        axis=1,
    )
    inp32 = inputs.astype(jnp.int32)

    out_emb, out_ctx, logits = _sc_deepwalk(
        target_table, context_table, idx_all, inp32)
    loss, mrr = _tc_loss(logits)
    return out_emb, out_ctx, loss[0, 0], mrr[0, 0]


# speedup vs baseline: 2.2932x; 2.2932x over previous
import functools

import jax
import jax.numpy as jnp
from jax import lax
from jax.experimental import pallas as pl
from jax.experimental.pallas import tpu as pltpu
from jax.experimental.pallas import tpu_sc as plsc

MAX_ID = 99999
DIM = 64
WALK_LEN = 5
WALK_NUM = 2
WIN = 2
NUM_NEGS = 5
BATCH = 1024
L = WALK_LEN + 1

_PAIR_SRC = []
_PAIR_POS = []
for _i in range(L):
    for _j in range(max(0, _i - WIN), min(L, _i + WIN + 1)):
        if _j != _i:
            _PAIR_SRC.append(_i)
            _PAIR_POS.append(_j)
NUM_PAIRS = len(_PAIR_SRC)

N = WALK_NUM * BATCH * NUM_PAIRS
CHUNK = 128
NCHUNKS = N // CHUNK
NW = 32
CPW = NCHUNKS // NW
OPW = BATCH // NW
NLANES = 16
NGROUPS = CHUNK // NLANES

_mesh = plsc.VectorSubcoreMesh(core_axis_name="c", subcore_axis_name="s")


@functools.partial(
    pl.kernel,
    out_type=[
        jax.ShapeDtypeStruct((BATCH, DIM), jnp.float32),
        jax.ShapeDtypeStruct((BATCH, DIM), jnp.float32),
        jax.ShapeDtypeStruct((NCHUNKS, 6, CHUNK), jnp.float32),
    ],
    mesh=_mesh,
    compiler_params=pltpu.CompilerParams(
        needs_layout_passes=False, use_tc_tiling_on_sc=False),
    scratch_types=[
        pltpu.VMEM((7, CHUNK), jnp.int32),
        pltpu.VMEM((CHUNK, DIM), jnp.float32),
        pltpu.VMEM((CHUNK, DIM), jnp.float32),
        pltpu.VMEM((NUM_NEGS, CHUNK, DIM), jnp.float32),
        pltpu.VMEM((6, CHUNK), jnp.float32),
        pltpu.VMEM((OPW,), jnp.int32),
        pltpu.VMEM((OPW, DIM), jnp.float32),
        pltpu.VMEM((OPW, DIM), jnp.float32),
        pltpu.SemaphoreType.DMA,
    ],
)
def _sc_deepwalk(tgt_hbm, ctx_hbm, idx_hbm, inp_hbm,
                 out_emb_hbm, out_ctx_hbm, logits_hbm,
                 idx_v, emb_v, pos_v, neg_v, lg_v, oidx_v, oemb_v, octx_v,
                 sem):
    nc = _mesh.num_cores
    wid = lax.axis_index("s") * nc + lax.axis_index("c")

    ob = wid * OPW
    pltpu.sync_copy(inp_hbm.at[pl.ds(ob, OPW)], oidx_v)
    h1 = pltpu.async_copy(tgt_hbm.at[oidx_v], oemb_v, sem)
    h2 = pltpu.async_copy(ctx_hbm.at[oidx_v], octx_v, sem)
    h1.wait()
    h2.wait()
    pltpu.sync_copy(oemb_v, out_emb_hbm.at[pl.ds(ob, OPW)])
    pltpu.sync_copy(octx_v, out_ctx_hbm.at[pl.ds(ob, OPW)])

    def chunk_body(ci, _):
        c = wid * CPW + ci
        pltpu.sync_copy(idx_hbm.at[c], idx_v)
        hs = [
            pltpu.async_copy(tgt_hbm.at[idx_v.at[0]], emb_v, sem),
            pltpu.async_copy(ctx_hbm.at[idx_v.at[1]], pos_v, sem),
        ]
        for k in range(NUM_NEGS):
            hs.append(
                pltpu.async_copy(ctx_hbm.at[idx_v.at[2 + k]], neg_v.at[k], sem))
        for h in hs:
            h.wait()

        for g in range(NGROUPS):
            rows = lax.iota(jnp.int32, NLANES) + g * NLANES

            def dstep(d, accs):
                col = jnp.full((NLANES,), 0, jnp.int32) + d
                ve = plsc.load_gather(emb_v, [rows, col])
                vp = plsc.load_gather(pos_v, [rows, col])
                out = [accs[0] + ve * vp]
                for k in range(NUM_NEGS):
                    vn = plsc.load_gather(neg_v.at[k], [rows, col])
                    out.append(accs[k + 1] + ve * vn)
                return tuple(out)

            zero = jnp.zeros((NLANES,), jnp.float32)
            accs = lax.fori_loop(0, DIM, dstep, (zero,) * 6)
            for j in range(6):
                lg_v[j, pl.ds(g * NLANES, NLANES)] = accs[j]

        pltpu.sync_copy(lg_v, logits_hbm.at[c])
        return 0

    lax.fori_loop(0, CPW, chunk_body, 0)


def _tc_loss_body(lg_ref, loss_ref, mrr_ref):
    x = lg_ref[...]
    pos = x[:, 0:1, :]
    negs = x[:, 1:, :]
    sp_pos = jnp.maximum(-pos, 0.0) + jnp.log1p(jnp.exp(-jnp.abs(pos)))
    sp_neg = jnp.maximum(negs, 0.0) + jnp.log1p(jnp.exp(-jnp.abs(negs)))
    loss_ref[0, 0] = jnp.sum(sp_pos) + jnp.sum(sp_neg)
    rank = 1.0 + jnp.sum((negs >= pos).astype(jnp.float32), axis=1)
    mrr_ref[0, 0] = jnp.sum(1.0 / rank) * (1.0 / N)


_tc_loss = pl.pallas_call(
    _tc_loss_body,
    out_shape=[
        jax.ShapeDtypeStruct((1, 1), jnp.float32),
        jax.ShapeDtypeStruct((1, 1), jnp.float32),
    ],
    out_specs=[
        pl.BlockSpec(memory_space=pltpu.SMEM),
        pl.BlockSpec(memory_space=pltpu.SMEM),
    ],
)


def kernel(inputs, paths, negs, target_table, context_table):
    src_i = jnp.asarray(_PAIR_SRC, jnp.int32)
    pos_i = jnp.asarray(_PAIR_POS, jnp.int32)
    src = jnp.take(paths, src_i, axis=2).reshape(N).astype(jnp.int32)
    pos = jnp.take(paths, pos_i, axis=2).reshape(N).astype(jnp.int32)
    ng = negs.reshape(N, NUM_NEGS).astype(jnp.int32)
    idx_all = jnp.concatenate(
        [
            src.reshape(NCHUNKS, 1, CHUNK),
            pos.reshape(NCHUNKS, 1, CHUNK),
            ng.reshape(NCHUNKS, CHUNK, NUM_NEGS).transpose(0, 2, 1),
        ],
        axis=1,
    )
    inp32 = inputs.astype(jnp.int32)

    out_emb, out_ctx, logits = _sc_deepwalk(
        target_table, context_table, idx_all, inp32)
    loss, mrr = _tc_loss(logits)
    return out_emb, out_ctx, loss[0, 0], mrr[0, 0]


# speedup vs baseline: 5.7866x; 2.5234x over previous
import functools

import jax
import jax.numpy as jnp
from jax import lax
from jax.experimental import pallas as pl
from jax.experimental.pallas import tpu as pltpu
from jax.experimental.pallas import tpu_sc as plsc

MAX_ID = 99999
DIM = 64
WALK_LEN = 5
WALK_NUM = 2
WIN = 2
NUM_NEGS = 5
BATCH = 1024
L = WALK_LEN + 1

_PAIR_SRC = []
_PAIR_POS = []
for _i in range(L):
    for _j in range(max(0, _i - WIN), min(L, _i + WIN + 1)):
        if _j != _i:
            _PAIR_SRC.append(_i)
            _PAIR_POS.append(_j)
NUM_PAIRS = len(_PAIR_SRC)

N = WALK_NUM * BATCH * NUM_PAIRS
CHUNK = 128
NCHUNKS = N // CHUNK
NW = 32
CPW = NCHUNKS // NW
OPW = BATCH // NW
NLANES = 16
NQ = DIM // NLANES

_mesh = plsc.VectorSubcoreMesh(core_axis_name="c", subcore_axis_name="s")


@functools.partial(
    pl.kernel,
    out_type=[
        jax.ShapeDtypeStruct((BATCH, DIM), jnp.float32),
        jax.ShapeDtypeStruct((BATCH, DIM), jnp.float32),
        jax.ShapeDtypeStruct((NCHUNKS, 6 * CHUNK), jnp.float32),
    ],
    mesh=_mesh,
    compiler_params=pltpu.CompilerParams(
        needs_layout_passes=False, use_tc_tiling_on_sc=False),
    scratch_types=[
        pltpu.VMEM((CPW, 7, CHUNK), jnp.int32),
        pltpu.VMEM((2, CHUNK, DIM), jnp.float32),
        pltpu.VMEM((2, CHUNK, DIM), jnp.float32),
        pltpu.VMEM((2, NUM_NEGS, CHUNK, DIM), jnp.float32),
        pltpu.VMEM((2, 6 * CHUNK), jnp.float32),
        pltpu.VMEM((OPW,), jnp.int32),
        pltpu.VMEM((OPW, DIM), jnp.float32),
        pltpu.VMEM((OPW, DIM), jnp.float32),
        pltpu.SemaphoreType.DMA,
        pltpu.SemaphoreType.DMA,
    ],
)
def _sc_deepwalk(tgt_hbm, ctx_hbm, idx_hbm, inp_hbm,
                 out_emb_hbm, out_ctx_hbm, logits_hbm,
                 idx_v, emb_v, pos_v, neg_v, lg_v, oidx_v, oemb_v, octx_v,
                 sem0, sem1):
    nc = _mesh.num_cores
    wid = lax.axis_index("s") * nc + lax.axis_index("c")
    sems = (sem0, sem1)

    ob = wid * OPW
    pltpu.sync_copy(inp_hbm.at[pl.ds(ob, OPW)], oidx_v)
    h1 = pltpu.async_copy(tgt_hbm.at[oidx_v], oemb_v, sem0)
    h2 = pltpu.async_copy(ctx_hbm.at[oidx_v], octx_v, sem1)
    pltpu.sync_copy(idx_hbm.at[wid], idx_v)
    h1.wait()
    h2.wait()
    pltpu.sync_copy(oemb_v, out_emb_hbm.at[pl.ds(ob, OPW)])
    pltpu.sync_copy(octx_v, out_ctx_hbm.at[pl.ds(ob, OPW)])

    def fire(ci, b):
        hs = [
            pltpu.async_copy(tgt_hbm.at[idx_v.at[ci, 0]], emb_v.at[b], sems[b]),
            pltpu.async_copy(ctx_hbm.at[idx_v.at[ci, 1]], pos_v.at[b], sems[b]),
        ]
        for k in range(NUM_NEGS):
            hs.append(pltpu.async_copy(
                ctx_hbm.at[idx_v.at[ci, 2 + k]], neg_v.at[b, k], sems[b]))
        return hs

    lane0 = lax.iota(jnp.int32, NLANES) == 0

    def compute(b):
        embb, posb, negb, lgb = emb_v.at[b], pos_v.at[b], neg_v.at[b], lg_v.at[b]

        @plsc.parallel_loop(0, CHUNK, 1, unroll=2)
        def row_body(r):
            rvec = jnp.full((NLANES,), 0, jnp.int32) + r
            ve = [embb[r, pl.ds(q * NLANES, NLANES)] for q in range(NQ)]
            vp = [posb[r, pl.ds(q * NLANES, NLANES)] for q in range(NQ)]
            pp = (ve[0] * vp[0] + ve[1] * vp[1]) + (ve[2] * vp[2] + ve[3] * vp[3])
            s = jnp.sum(pp)
            plsc.store_scatter(lgb, [rvec],
                               jnp.full((NLANES,), 0.0, jnp.float32) + s,
                               mask=lane0)
            for k in range(NUM_NEGS):
                vn = [negb[k, r, pl.ds(q * NLANES, NLANES)] for q in range(NQ)]
                nn = (ve[0] * vn[0] + ve[1] * vn[1]) + (ve[2] * vn[2] + ve[3] * vn[3])
                s = jnp.sum(nn)
                plsc.store_scatter(lgb, [rvec + (1 + k) * CHUNK],
                                   jnp.full((NLANES,), 0.0, jnp.float32) + s,
                                   mask=lane0)

    hs = fire(0, 0)
    for ci in range(CPW):
        b = ci % 2
        for h in hs:
            h.wait()
        if ci + 1 < CPW:
            hs = fire(ci + 1, 1 - b)
        compute(b)
        pltpu.sync_copy(lg_v.at[b], logits_hbm.at[wid * CPW + ci])


def _tc_loss_body(lg_ref, loss_ref, mrr_ref):
    x = lg_ref[...]
    pos = x[:, jnp.newaxis, 0:CHUNK]
    negs = jnp.stack(
        [x[:, (1 + k) * CHUNK:(2 + k) * CHUNK] for k in range(NUM_NEGS)],
        axis=1)
    sp_pos = jnp.maximum(-pos, 0.0) + jnp.log1p(jnp.exp(-jnp.abs(pos)))
    sp_neg = jnp.maximum(negs, 0.0) + jnp.log1p(jnp.exp(-jnp.abs(negs)))
    loss_ref[0, 0] = jnp.sum(sp_pos) + jnp.sum(sp_neg)
    rank = 1.0 + jnp.sum((negs >= pos).astype(jnp.float32), axis=1)
    mrr_ref[0, 0] = jnp.sum(1.0 / rank) * (1.0 / N)


_tc_loss = pl.pallas_call(
    _tc_loss_body,
    out_shape=[
        jax.ShapeDtypeStruct((1, 1), jnp.float32),
        jax.ShapeDtypeStruct((1, 1), jnp.float32),
    ],
    out_specs=[
        pl.BlockSpec(memory_space=pltpu.SMEM),
        pl.BlockSpec(memory_space=pltpu.SMEM),
    ],
)


def kernel(inputs, paths, negs, target_table, context_table):
    src_i = jnp.asarray(_PAIR_SRC, jnp.int32)
    pos_i = jnp.asarray(_PAIR_POS, jnp.int32)
    src = jnp.take(paths, src_i, axis=2).reshape(N).astype(jnp.int32)
    pos = jnp.take(paths, pos_i, axis=2).reshape(N).astype(jnp.int32)
    ng = negs.reshape(N, NUM_NEGS).astype(jnp.int32)
    idx_all = jnp.concatenate(
        [
            src.reshape(NCHUNKS, 1, CHUNK),
            pos.reshape(NCHUNKS, 1, CHUNK),
            ng.reshape(NCHUNKS, CHUNK, NUM_NEGS).transpose(0, 2, 1),
        ],
        axis=1,
    ).reshape(NW, CPW, 7, CHUNK)
    inp32 = inputs.astype(jnp.int32)

    out_emb, out_ctx, logits = _sc_deepwalk(
        target_table, context_table, idx_all, inp32)
    loss, mrr = _tc_loss(logits)
    return out_emb, out_ctx, loss[0, 0], mrr[0, 0]
